# Initial kernel scaffold; baseline (speedup 1.0000x reference)
import jax
import jax.numpy as jnp
from jax.experimental import pallas as pl


def kernel(x, edge_index, W1, a_s1, a_d1, b1, g1, be1, W2, a_s2, a_d2, b2, g2, be2, W3, a_s3, a_d3, b3, g3, be3, W4, a_s4, a_d4, b4):
    raise NotImplementedError("write your pallas kernel here")


# speedup vs baseline: 2.1312x; 2.1312x over previous
import functools
import jax
import jax.numpy as jnp
from jax.experimental import pallas as pl

H = 8
N_NODES = 10000
NB = 1000


def _proj_kernel(x_ref, w_ref, asf_ref, adf_ref, h_ref, asrc_ref, adst_ref,
                 *, d_head, n_heads):
    c4 = pl.program_id(1)
    acc = jnp.dot(x_ref[...], w_ref[...], preferred_element_type=jnp.float32)
    s_up = jnp.zeros((x_ref.shape[0], n_heads), jnp.float32)
    d_up = jnp.zeros((x_ref.shape[0], n_heads), jnp.float32)
    for i in range(4):
        blk = acc[:, i * 64:(i + 1) * 64]
        h_ref[i] = blk
        col = c4 * 4 + i
        hd = (col * 64) // d_head
        mask = (jax.lax.broadcasted_iota(jnp.int32, (1, n_heads), 1) == hd
                ).astype(jnp.float32)
        sv = jnp.sum(blk * asf_ref[0, i * 64:(i + 1) * 64][None, :], axis=1)
        dv = jnp.sum(blk * adf_ref[0, i * 64:(i + 1) * 64][None, :], axis=1)
        s_up = s_up + sv[:, None] * mask
        d_up = d_up + dv[:, None] * mask

    @pl.when(c4 == 0)
    def _():
        asrc_ref[...] = jnp.zeros_like(asrc_ref)
        adst_ref[...] = jnp.zeros_like(adst_ref)

    asrc_ref[...] += s_up
    adst_ref[...] += d_up


def _proj(x, w, a_s, a_d, d_head, n_heads):
    din = x.shape[1]
    hd_total = w.shape[1]
    n_chunks = hd_total // 64
    asf = a_s.reshape(1, hd_total)
    adf = a_d.reshape(1, hd_total)
    grid = (N_NODES // NB, n_chunks // 4)
    return pl.pallas_call(
        functools.partial(_proj_kernel, d_head=d_head, n_heads=n_heads),
        grid=grid,
        in_specs=[
            pl.BlockSpec((NB, din), lambda n, c: (n, 0)),
            pl.BlockSpec((din, 256), lambda n, c: (0, c)),
            pl.BlockSpec((1, 256), lambda n, c: (0, c)),
            pl.BlockSpec((1, 256), lambda n, c: (0, c)),
        ],
        out_specs=[
            pl.BlockSpec((4, NB, 64), lambda n, c: (c, n, 0)),
            pl.BlockSpec((NB, n_heads), lambda n, c: (n, 0)),
            pl.BlockSpec((NB, n_heads), lambda n, c: (n, 0)),
        ],
        out_shape=[
            jax.ShapeDtypeStruct((n_chunks, N_NODES, 64), jnp.float32),
            jax.ShapeDtypeStruct((N_NODES, n_heads), jnp.float32),
            jax.ShapeDtypeStruct((N_NODES, n_heads), jnp.float32),
        ],
    )(x, w, asf, adf)


def _post_kernel(outw_ref, den_ref, b_ref, g_ref, be_ref, y_ref,
                 *, d_head, n_heads, do_ln):
    den = den_ref[0] + den_ref[1] + 1e-16
    nb = outw_ref.shape[0]
    hd_total = outw_ref.shape[1]
    den_e = jnp.broadcast_to(den[:, :, None], (nb, n_heads, d_head)
                             ).reshape(nb, hd_total)
    z = outw_ref[...] / den_e + b_ref[0][None, :]
    if do_ln:
        z = jnp.maximum(z, 0.0)
        mu = jnp.mean(z, axis=1, keepdims=True)
        var = jnp.mean((z - mu) ** 2, axis=1, keepdims=True)
        z = (z - mu) * jax.lax.rsqrt(var + 1e-5) * g_ref[0][None, :] \
            + be_ref[0][None, :]
    y_ref[...] = z


NB_POST = 400


def _post(outw, den2, b, g, be, d_head, n_heads, do_ln):
    hd_total = outw.shape[1]
    return pl.pallas_call(
        functools.partial(_post_kernel, d_head=d_head, n_heads=n_heads,
                          do_ln=do_ln),
        grid=(N_NODES // NB_POST,),
        in_specs=[
            pl.BlockSpec((NB_POST, hd_total), lambda n: (n, 0)),
            pl.BlockSpec((2, NB_POST, n_heads), lambda n: (0, n, 0)),
            pl.BlockSpec((1, hd_total), lambda n: (0, 0)),
            pl.BlockSpec((1, hd_total), lambda n: (0, 0)),
            pl.BlockSpec((1, hd_total), lambda n: (0, 0)),
        ],
        out_specs=pl.BlockSpec((NB_POST, hd_total), lambda n: (n, 0)),
        out_shape=jax.ShapeDtypeStruct((N_NODES, hd_total), jnp.float32),
    )(outw, den2, b.reshape(1, -1), g.reshape(1, -1), be.reshape(1, -1))


def _edge_temp(h3d, asrc, adst, src, dst, valid, d_head, n_heads):
    n_chunks = h3d.shape[0]
    hd_total = n_chunks * 64
    e = asrc[src] + adst[dst]
    e = jnp.where(e > 0, e, 0.2 * e)
    ew = jnp.where(valid[:, None], jnp.exp(e), 0.0)
    den = jax.ops.segment_sum(ew, dst, num_segments=N_NODES)
    h2d = jnp.moveaxis(h3d, 0, 1).reshape(N_NODES, hd_total)
    rows = h2d[src].reshape(-1, n_heads, d_head) * ew[:, :, None]
    outw = jax.ops.segment_sum(rows.reshape(-1, hd_total), dst,
                               num_segments=N_NODES)
    den2 = jnp.stack([den, jnp.zeros_like(den)])
    return outw, den2


def _gat_layer(x, src, dst, valid, w, a_s, a_d, b, g, be, d_head, n_heads,
               do_ln):
    h3d, asrc, adst = _proj(x, w, a_s, a_d, d_head, n_heads)
    outw, den2 = _edge_temp(h3d, asrc, adst, src, dst, valid, d_head, n_heads)
    return _post(outw, den2, b, g, be, d_head, n_heads, do_ln)


def kernel(x, edge_index, W1, a_s1, a_d1, b1, g1, be1, W2, a_s2, a_d2, b2,
           g2, be2, W3, a_s3, a_d3, b3, g3, be3, W4, a_s4, a_d4, b4):
    n = x.shape[0]
    loop = jnp.arange(n, dtype=edge_index.dtype)
    src = jnp.concatenate([edge_index[0], loop])
    dst = jnp.concatenate([edge_index[1], loop])
    e_real = src.shape[0]
    e_pad = 172032
    src = jnp.pad(src, (0, e_pad - e_real))
    dst = jnp.pad(dst, (0, e_pad - e_real))
    valid = jnp.arange(e_pad) < e_real

    h = _gat_layer(x, src, dst, valid, W1, a_s1, a_d1, b1, g1, be1, 448, H,
                   True)
    h = _gat_layer(h, src, dst, valid, W2, a_s2, a_d2, b2, g2, be2, 384, H,
                   True)
    h = _gat_layer(h, src, dst, valid, W3, a_s3, a_d3, b3, g3, be3, 256, H,
                   True)
    w4p = jnp.pad(W4, ((0, 0), (0, 1024 - W4.shape[1])))
    as4p = jnp.pad(a_s4, ((0, 0), (0, 1024 - a_s4.shape[1])))
    ad4p = jnp.pad(a_d4, ((0, 0), (0, 1024 - a_d4.shape[1])))
    b4p = jnp.pad(b4, (0, 1024 - b4.shape[0]))
    out = _gat_layer(h, src, dst, valid, w4p, as4p, ad4p, b4p, b4p, b4p,
                     1024, 1, False)
    return out[:, :W4.shape[1]]


# speedup vs baseline: 4.4533x; 2.0896x over previous
import functools
import jax
import jax.numpy as jnp
from jax.experimental import pallas as pl
from jax.experimental.pallas import tpu as pltpu
from jax.experimental.pallas import tpu_sc as plsc

H = 8
N_NODES = 10000
NB = 1000
NB_POST = 400
E_PAD = 172032
EB = 256
EBS = 128
DUMMY = N_NODES
NEG_BIG = -1e30
STRIPE = 640
NROWS = 10240


def _proj_kernel(x_ref, w_ref, asf_ref, adf_ref, h_ref, asrc_ref, adst_ref,
                 *, d_head):
    c2 = pl.program_id(1)
    acc = jnp.dot(x_ref[...], w_ref[...], preferred_element_type=jnp.float32)
    s_up = jnp.zeros((x_ref.shape[0], 128), jnp.float32)
    d_up = jnp.zeros((x_ref.shape[0], 128), jnp.float32)
    for i in range(4):
        col = c2 * 4 + i
        hd = (col * 64) // d_head
        mask = (jax.lax.broadcasted_iota(jnp.int32, (1, 128), 1) == hd
                ).astype(jnp.float32)
        blk = acc[:, i * 64:(i + 1) * 64]
        sv = jnp.sum(blk * asf_ref[0, i * 64:(i + 1) * 64][None, :], axis=1)
        dv = jnp.sum(blk * adf_ref[0, i * 64:(i + 1) * 64][None, :], axis=1)
        s_up = s_up + sv[:, None] * mask
        d_up = d_up + dv[:, None] * mask
    for i in range(2):
        h_ref[i] = acc[:, i * 128:(i + 1) * 128]

    @pl.when(c2 == 0)
    def _():
        asrc_ref[...] = jnp.zeros_like(asrc_ref)
        adst_ref[...] = jnp.zeros_like(adst_ref)

    asrc_ref[...] += s_up
    adst_ref[...] += d_up


def _proj(x, w, a_s, a_d, d_head):
    din = x.shape[1]
    hd_total = w.shape[1]
    c2 = hd_total // 128
    asf = a_s.reshape(1, hd_total)
    adf = a_d.reshape(1, hd_total)
    grid = (N_NODES // NB, c2 // 2)
    return pl.pallas_call(
        functools.partial(_proj_kernel, d_head=d_head),
        grid=grid,
        in_specs=[
            pl.BlockSpec((NB, din), lambda n, c: (n, 0)),
            pl.BlockSpec((din, 256), lambda n, c: (0, c)),
            pl.BlockSpec((1, 256), lambda n, c: (0, c)),
            pl.BlockSpec((1, 256), lambda n, c: (0, c)),
        ],
        out_specs=[
            pl.BlockSpec((2, NB, 128), lambda n, c: (c, n, 0)),
            pl.BlockSpec((NB, 128), lambda n, c: (n, 0)),
            pl.BlockSpec((NB, 128), lambda n, c: (n, 0)),
        ],
        out_shape=[
            jax.ShapeDtypeStruct((c2, N_NODES, 128), jnp.float32),
            jax.ShapeDtypeStruct((N_NODES, 128), jnp.float32),
            jax.ShapeDtypeStruct((N_NODES, 128), jnp.float32),
        ],
    )(x, w, asf, adf)


def _post_kernel(outw_ref, den_ref, b_ref, g_ref, be_ref, y_ref,
                 *, d_head, n_heads, do_ln):
    den = den_ref[0, :, :n_heads] + den_ref[1, :, :n_heads] + 1e-16
    nb = outw_ref.shape[0]
    hd_total = y_ref.shape[1]
    den_e = jnp.broadcast_to(den[:, :, None], (nb, n_heads, d_head)
                             ).reshape(nb, hd_total)
    z = outw_ref[...] / den_e + b_ref[0][None, :]
    if do_ln:
        z = jnp.maximum(z, 0.0)
        mu = jnp.mean(z, axis=1, keepdims=True)
        var = jnp.mean((z - mu) ** 2, axis=1, keepdims=True)
        z = (z - mu) * jax.lax.rsqrt(var + 1e-5) * g_ref[0][None, :] \
            + be_ref[0][None, :]
    y_ref[...] = z


def _post(outw, denp, b, g, be, d_head, n_heads, do_ln):
    hd_total = outw.shape[1]
    return pl.pallas_call(
        functools.partial(_post_kernel, d_head=d_head, n_heads=n_heads,
                          do_ln=do_ln),
        grid=(N_NODES // NB_POST,),
        in_specs=[
            pl.BlockSpec((NB_POST, hd_total), lambda n: (n, 0)),
            pl.BlockSpec((2, NB_POST, 128), lambda n: (0, n, 0)),
            pl.BlockSpec((1, hd_total), lambda n: (0, 0)),
            pl.BlockSpec((1, hd_total), lambda n: (0, 0)),
            pl.BlockSpec((1, hd_total), lambda n: (0, 0)),
        ],
        out_specs=pl.BlockSpec((NB_POST, hd_total), lambda n: (n, 0)),
        out_shape=jax.ShapeDtypeStruct((N_NODES, hd_total), jnp.float32),
    )(outw, denp, b.reshape(1, -1), g.reshape(1, -1), be.reshape(1, -1))


def _sc_mesh():
    return plsc.VectorSubcoreMesh(core_axis_name="c", subcore_axis_name="s")


def _sc_softmax(asrc, adst_p, src, dst):
    per_w = E_PAD // 32
    n_batches = per_w // EBS

    @functools.partial(
        pl.kernel, mesh=_sc_mesh(),
        out_type=[
            jax.ShapeDtypeStruct((E_PAD * 16,), jnp.float32),
            jax.ShapeDtypeStruct((2 * NROWS, 128), jnp.float32),
        ],
        scratch_types=[
            pltpu.VMEM((EBS,), jnp.int32),
            pltpu.VMEM((1, EBS), jnp.int32),
            pltpu.VMEM((EBS, 128), jnp.float32),
            pltpu.VMEM((EBS, 128), jnp.float32),
            pltpu.VMEM((EBS * 16,), jnp.float32),
            pltpu.VMEM_SHARED((NROWS, 128), jnp.float32),
        ],
    )
    def k(asrc_hbm, adst_hbm, src_hbm, dst_hbm, zden_hbm, ew_hbm, denp_hbm,
          sidx, didx, arows, brows, ewf, den_sh):
        core = jax.lax.axis_index("c")
        sub = jax.lax.axis_index("s")
        wid = sub * 2 + core
        pltpu.sync_copy(zden_hbm, den_sh.at[pl.ds(sub * STRIPE, STRIPE)])
        plsc.subcore_barrier()
        base_w = wid * per_w

        @pl.loop(0, n_batches)
        def _(b):
            base = base_w + b * EBS
            pltpu.sync_copy(src_hbm.at[pl.ds(base, EBS)], sidx)
            pltpu.sync_copy(dst_hbm.at[pl.ds(base, EBS)], didx.at[0])
            pltpu.sync_copy(asrc_hbm.at[sidx], arows)
            pltpu.sync_copy(adst_hbm.at[didx.at[0]], brows)

            @pl.loop(0, EBS)
            def _(i):
                v = arows[i, pl.ds(0, 16)] + brows[i, pl.ds(0, 16)]
                v = jnp.where(v > 0, v, 0.2 * v)
                v = jnp.exp(v)
                arows[i, pl.ds(0, 16)] = v
                ewf[pl.ds(i * 16, 16)] = v

            pltpu.sync_copy(ewf, ew_hbm.at[pl.ds(base * 16, EBS * 16)])
            pltpu.sync_copy(arows, den_sh.at[didx.at[0]], add=True)

        plsc.subcore_barrier()
        pltpu.sync_copy(
            den_sh.at[pl.ds(sub * STRIPE, STRIPE)],
            denp_hbm.at[pl.ds(core * NROWS + sub * STRIPE, STRIPE)])

    zden = jnp.zeros((STRIPE, 128), jnp.float32)
    ew, denp = k(asrc, adst_p, src, dst, zden)
    return ew, denp.reshape(2, NROWS, 128)[:, :N_NODES]


def _sc_spmm(h2d, src, dst, ew, c2, d_head):
    per_s = E_PAD // 16
    nb2 = per_s // EB
    half = c2 // 2
    hd_total = c2 * 128

    @functools.partial(
        pl.kernel, mesh=_sc_mesh(),
        out_type=jax.ShapeDtypeStruct((NROWS, hd_total), jnp.float32),
        scratch_types=[
            pltpu.VMEM((EB,), jnp.int32),
            pltpu.VMEM((1, EB), jnp.int32),
            pltpu.VMEM((EB, 128), jnp.float32),
            pltpu.VMEM((EB * 16,), jnp.float32),
            pltpu.VMEM_SHARED((NROWS, 128), jnp.float32),
        ],
    )
    def k(h_hbm, src_hbm, dst_hbm, ew_hbm, zacc_hbm, out_hbm,
          gidx, didx, rows, ewbf, acc):
        core = jax.lax.axis_index("c")
        sub = jax.lax.axis_index("s")

        @pl.loop(0, half)
        def _(j):
            c = j * 2 + core
            hd_lo = (c * 128) // d_head
            hd_hi = (c * 128 + 64) // d_head
            cbase = c * N_NODES
            pltpu.sync_copy(zacc_hbm, acc.at[pl.ds(sub * STRIPE, STRIPE)])
            plsc.subcore_barrier()

            @pl.loop(0, nb2)
            def _(b):
                base = sub * per_s + b * EB
                pltpu.sync_copy(src_hbm.at[pl.ds(base, EB)], gidx)
                pltpu.sync_copy(dst_hbm.at[pl.ds(base, EB)], didx.at[0])
                pltpu.sync_copy(ew_hbm.at[pl.ds(base * 16, EB * 16)], ewbf)

                @pl.loop(0, EB, step=16)
                def _(i):
                    gidx[pl.ds(i, 16)] = gidx[pl.ds(i, 16)] + cbase

                pltpu.sync_copy(h_hbm.at[gidx], rows)

                @pl.loop(0, EB)
                def _(i):
                    wv = ewbf[pl.ds(i * 16, 16)]
                    wlo = wv[jnp.full((16,), hd_lo, jnp.int32)]
                    whi = wv[jnp.full((16,), hd_hi, jnp.int32)]
                    for jc in range(4):
                        sl = pl.ds(jc * 16, 16)
                        rows[i, sl] = rows[i, sl] * wlo
                    for jc in range(4, 8):
                        sl = pl.ds(jc * 16, 16)
                        rows[i, sl] = rows[i, sl] * whi

                pltpu.sync_copy(rows, acc.at[didx.at[0]], add=True)

            plsc.subcore_barrier()
            pltpu.sync_copy(
                acc.at[pl.ds(sub * STRIPE, STRIPE)],
                out_hbm.at[pl.ds(sub * STRIPE, STRIPE), pl.ds(c * 128, 128)])

    zacc = jnp.zeros((STRIPE, 128), jnp.float32)
    return k(h2d, src, dst, ew, zacc)


def _gat_layer(x, src, dst, w, a_s, a_d, b, g, be, d_head, n_heads, do_ln):
    h3d, asrc, adst = _proj(x, w, a_s, a_d, d_head)
    c2 = h3d.shape[0]
    adst_p = jnp.concatenate(
        [adst, jnp.full((16, 128), NEG_BIG, jnp.float32)])
    ew, denp = _sc_softmax(asrc, adst_p, src, dst)
    h2d = h3d.reshape(c2 * N_NODES, 128)
    outw = _sc_spmm(h2d, src, dst, ew, c2, d_head)
    return _post(outw[:N_NODES], denp, b, g, be, d_head, n_heads, do_ln)


def kernel(x, edge_index, W1, a_s1, a_d1, b1, g1, be1, W2, a_s2, a_d2, b2,
           g2, be2, W3, a_s3, a_d3, b3, g3, be3, W4, a_s4, a_d4, b4):
    n = x.shape[0]
    loop = jnp.arange(n, dtype=edge_index.dtype)
    src = jnp.concatenate([edge_index[0], loop])
    dst = jnp.concatenate([edge_index[1], loop])
    e_real = src.shape[0]
    src = jnp.pad(src, (0, E_PAD - e_real))
    dst = jnp.pad(dst, (0, E_PAD - e_real), constant_values=DUMMY)

    h = _gat_layer(x, src, dst, W1, a_s1, a_d1, b1, g1, be1, 448, H, True)
    h = _gat_layer(h, src, dst, W2, a_s2, a_d2, b2, g2, be2, 384, H, True)
    h = _gat_layer(h, src, dst, W3, a_s3, a_d3, b3, g3, be3, 256, H, True)
    w4p = jnp.pad(W4, ((0, 0), (0, 1024 - W4.shape[1])))
    as4p = jnp.pad(a_s4, ((0, 0), (0, 1024 - a_s4.shape[1])))
    ad4p = jnp.pad(a_d4, ((0, 0), (0, 1024 - a_d4.shape[1])))
    b4p = jnp.pad(b4, (0, 1024 - b4.shape[0]))
    out = _gat_layer(h, src, dst, w4p, as4p, ad4p, b4p, b4p, b4p,
                     1024, 1, False)
    return out[:, :W4.shape[1]]


# speedup vs baseline: 4.4772x; 1.0054x over previous
import functools
import jax
import jax.numpy as jnp
from jax.experimental import pallas as pl
from jax.experimental.pallas import tpu as pltpu
from jax.experimental.pallas import tpu_sc as plsc

H = 8
N_NODES = 10000
NB = 1000
NB_POST = 400
E_PAD = 172032
EB = 256
EBS = 128
DUMMY = N_NODES
NEG_BIG = -1e30
STRIPE = 640
NROWS = 10240


def _proj_kernel(x_ref, w_ref, asf_ref, adf_ref, h_ref, asrc_ref, adst_ref,
                 *, d_head):
    c2 = pl.program_id(1)
    acc = jnp.dot(x_ref[...], w_ref[...], preferred_element_type=jnp.float32)
    s_up = jnp.zeros((x_ref.shape[0], 128), jnp.float32)
    d_up = jnp.zeros((x_ref.shape[0], 128), jnp.float32)
    for i in range(4):
        col = c2 * 4 + i
        hd = (col * 64) // d_head
        mask = (jax.lax.broadcasted_iota(jnp.int32, (1, 128), 1) == hd
                ).astype(jnp.float32)
        blk = acc[:, i * 64:(i + 1) * 64]
        sv = jnp.sum(blk * asf_ref[0, i * 64:(i + 1) * 64][None, :], axis=1)
        dv = jnp.sum(blk * adf_ref[0, i * 64:(i + 1) * 64][None, :], axis=1)
        s_up = s_up + sv[:, None] * mask
        d_up = d_up + dv[:, None] * mask
    for i in range(2):
        h_ref[i] = acc[:, i * 128:(i + 1) * 128]

    @pl.when(c2 == 0)
    def _():
        asrc_ref[...] = jnp.zeros_like(asrc_ref)
        adst_ref[...] = jnp.zeros_like(adst_ref)

    asrc_ref[...] += s_up
    adst_ref[...] += d_up


def _proj(x, w, a_s, a_d, d_head):
    din = x.shape[1]
    hd_total = w.shape[1]
    c2 = hd_total // 128
    asf = a_s.reshape(1, hd_total)
    adf = a_d.reshape(1, hd_total)
    grid = (N_NODES // NB, c2 // 2)
    return pl.pallas_call(
        functools.partial(_proj_kernel, d_head=d_head),
        grid=grid,
        in_specs=[
            pl.BlockSpec((NB, din), lambda n, c: (n, 0)),
            pl.BlockSpec((din, 256), lambda n, c: (0, c)),
            pl.BlockSpec((1, 256), lambda n, c: (0, c)),
            pl.BlockSpec((1, 256), lambda n, c: (0, c)),
        ],
        out_specs=[
            pl.BlockSpec((2, NB, 128), lambda n, c: (c, n, 0)),
            pl.BlockSpec((NB, 128), lambda n, c: (n, 0)),
            pl.BlockSpec((NB, 128), lambda n, c: (n, 0)),
        ],
        out_shape=[
            jax.ShapeDtypeStruct((c2, N_NODES, 128), jnp.float32),
            jax.ShapeDtypeStruct((N_NODES, 128), jnp.float32),
            jax.ShapeDtypeStruct((N_NODES, 128), jnp.float32),
        ],
    )(x, w, asf, adf)


def _post_kernel(outw_ref, den_ref, b_ref, g_ref, be_ref, y_ref,
                 *, d_head, n_heads, do_ln):
    den = den_ref[0, :, :n_heads] + den_ref[1, :, :n_heads] + 1e-16
    nb = outw_ref.shape[0]
    hd_total = y_ref.shape[1]
    den_e = jnp.broadcast_to(den[:, :, None], (nb, n_heads, d_head)
                             ).reshape(nb, hd_total)
    z = outw_ref[...] / den_e + b_ref[0][None, :]
    if do_ln:
        z = jnp.maximum(z, 0.0)
        mu = jnp.mean(z, axis=1, keepdims=True)
        var = jnp.mean((z - mu) ** 2, axis=1, keepdims=True)
        z = (z - mu) * jax.lax.rsqrt(var + 1e-5) * g_ref[0][None, :] \
            + be_ref[0][None, :]
    y_ref[...] = z


def _post(outw, denp, b, g, be, d_head, n_heads, do_ln):
    hd_total = outw.shape[1]
    return pl.pallas_call(
        functools.partial(_post_kernel, d_head=d_head, n_heads=n_heads,
                          do_ln=do_ln),
        grid=(N_NODES // NB_POST,),
        in_specs=[
            pl.BlockSpec((NB_POST, hd_total), lambda n: (n, 0)),
            pl.BlockSpec((2, NB_POST, 128), lambda n: (0, n, 0)),
            pl.BlockSpec((1, hd_total), lambda n: (0, 0)),
            pl.BlockSpec((1, hd_total), lambda n: (0, 0)),
            pl.BlockSpec((1, hd_total), lambda n: (0, 0)),
        ],
        out_specs=pl.BlockSpec((NB_POST, hd_total), lambda n: (n, 0)),
        out_shape=jax.ShapeDtypeStruct((N_NODES, hd_total), jnp.float32),
    )(outw, denp, b.reshape(1, -1), g.reshape(1, -1), be.reshape(1, -1))


def _sc_mesh():
    return plsc.VectorSubcoreMesh(core_axis_name="c", subcore_axis_name="s")


def _sc_softmax(asrc, adst_p, src, dst):
    per_w = E_PAD // 32
    n_batches = per_w // EBS

    @functools.partial(
        pl.kernel, mesh=_sc_mesh(),
        out_type=[
            jax.ShapeDtypeStruct((E_PAD * 16,), jnp.float32),
            jax.ShapeDtypeStruct((2 * NROWS, 128), jnp.float32),
        ],
        scratch_types=[
            pltpu.VMEM((EBS,), jnp.int32),
            pltpu.VMEM((1, EBS), jnp.int32),
            pltpu.VMEM((EBS, 128), jnp.float32),
            pltpu.VMEM((EBS, 128), jnp.float32),
            pltpu.VMEM((EBS * 16,), jnp.float32),
            pltpu.VMEM_SHARED((NROWS, 128), jnp.float32),
        ],
    )
    def k(asrc_hbm, adst_hbm, src_hbm, dst_hbm, zden_hbm, ew_hbm, denp_hbm,
          sidx, didx, arows, brows, ewf, den_sh):
        core = jax.lax.axis_index("c")
        sub = jax.lax.axis_index("s")
        wid = sub * 2 + core
        pltpu.sync_copy(zden_hbm, den_sh.at[pl.ds(sub * STRIPE, STRIPE)])
        plsc.subcore_barrier()
        base_w = wid * per_w

        @pl.loop(0, n_batches)
        def _(b):
            base = base_w + b * EBS
            pltpu.sync_copy(src_hbm.at[pl.ds(base, EBS)], sidx)
            pltpu.sync_copy(dst_hbm.at[pl.ds(base, EBS)], didx.at[0])
            pltpu.sync_copy(asrc_hbm.at[sidx], arows)
            pltpu.sync_copy(adst_hbm.at[didx.at[0]], brows)

            @pl.loop(0, EBS)
            def _(i):
                v = arows[i, pl.ds(0, 16)] + brows[i, pl.ds(0, 16)]
                v = jnp.where(v > 0, v, 0.2 * v)
                v = jnp.exp(v)
                arows[i, pl.ds(0, 16)] = v
                ewf[pl.ds(i * 16, 16)] = v

            pltpu.sync_copy(ewf, ew_hbm.at[pl.ds(base * 16, EBS * 16)])
            pltpu.sync_copy(arows, den_sh.at[didx.at[0]], add=True)

        plsc.subcore_barrier()
        pltpu.sync_copy(
            den_sh.at[pl.ds(sub * STRIPE, STRIPE)],
            denp_hbm.at[pl.ds(core * NROWS + sub * STRIPE, STRIPE)])

    zden = jnp.zeros((STRIPE, 128), jnp.float32)
    ew, denp = k(asrc, adst_p, src, dst, zden)
    return ew, denp.reshape(2, NROWS, 128)


def _sc_spmm(h2d, src, dst, ew, c2, d_head):
    per_s = E_PAD // 16
    nb2 = per_s // EB
    half = c2 // 2
    hd_total = c2 * 128

    @functools.partial(
        pl.kernel, mesh=_sc_mesh(),
        out_type=jax.ShapeDtypeStruct((NROWS, hd_total), jnp.float32),
        scratch_types=[
            pltpu.VMEM((EB,), jnp.int32),
            pltpu.VMEM((1, EB), jnp.int32),
            pltpu.VMEM((EB, 128), jnp.float32),
            pltpu.VMEM((EB * 16,), jnp.float32),
            pltpu.VMEM_SHARED((NROWS, 128), jnp.float32),
        ],
    )
    def k(h_hbm, src_hbm, dst_hbm, ew_hbm, zacc_hbm, out_hbm,
          gidx, didx, rows, ewbf, acc):
        core = jax.lax.axis_index("c")
        sub = jax.lax.axis_index("s")

        @pl.loop(0, half)
        def _(j):
            c = j * 2 + core
            hd_lo = (c * 128) // d_head
            hd_hi = (c * 128 + 64) // d_head
            cbase = c * N_NODES
            pltpu.sync_copy(zacc_hbm, acc.at[pl.ds(sub * STRIPE, STRIPE)])
            plsc.subcore_barrier()

            @pl.loop(0, nb2)
            def _(b):
                base = sub * per_s + b * EB
                pltpu.sync_copy(src_hbm.at[pl.ds(base, EB)], gidx)
                pltpu.sync_copy(dst_hbm.at[pl.ds(base, EB)], didx.at[0])
                pltpu.sync_copy(ew_hbm.at[pl.ds(base * 16, EB * 16)], ewbf)

                @pl.loop(0, EB, step=16)
                def _(i):
                    gidx[pl.ds(i, 16)] = gidx[pl.ds(i, 16)] + cbase

                pltpu.sync_copy(h_hbm.at[gidx], rows)

                @pl.loop(0, EB)
                def _(i):
                    wv = ewbf[pl.ds(i * 16, 16)]
                    wlo = wv[jnp.full((16,), hd_lo, jnp.int32)]
                    whi = wv[jnp.full((16,), hd_hi, jnp.int32)]
                    for jc in range(4):
                        sl = pl.ds(jc * 16, 16)
                        rows[i, sl] = rows[i, sl] * wlo
                    for jc in range(4, 8):
                        sl = pl.ds(jc * 16, 16)
                        rows[i, sl] = rows[i, sl] * whi

                pltpu.sync_copy(rows, acc.at[didx.at[0]], add=True)

            plsc.subcore_barrier()
            pltpu.sync_copy(
                acc.at[pl.ds(sub * STRIPE, STRIPE)],
                out_hbm.at[pl.ds(sub * STRIPE, STRIPE), pl.ds(c * 128, 128)])

    zacc = jnp.zeros((STRIPE, 128), jnp.float32)
    return k(h2d, src, dst, ew, zacc)


def _gat_layer(x, src, dst, w, a_s, a_d, b, g, be, d_head, n_heads, do_ln):
    h3d, asrc, adst = _proj(x, w, a_s, a_d, d_head)
    c2 = h3d.shape[0]
    adst_p = jnp.concatenate(
        [adst, jnp.full((16, 128), NEG_BIG, jnp.float32)])
    ew, denp = _sc_softmax(asrc, adst_p, src, dst)
    h2d = h3d.reshape(c2 * N_NODES, 128)
    outw = _sc_spmm(h2d, src, dst, ew, c2, d_head)
    return _post(outw, denp, b, g, be, d_head, n_heads, do_ln)


def kernel(x, edge_index, W1, a_s1, a_d1, b1, g1, be1, W2, a_s2, a_d2, b2,
           g2, be2, W3, a_s3, a_d3, b3, g3, be3, W4, a_s4, a_d4, b4):
    n = x.shape[0]
    loop = jnp.arange(n, dtype=edge_index.dtype)
    src = jnp.concatenate([edge_index[0], loop])
    dst = jnp.concatenate([edge_index[1], loop])
    e_real = src.shape[0]
    src = jnp.pad(src, (0, E_PAD - e_real))
    dst = jnp.pad(dst, (0, E_PAD - e_real), constant_values=DUMMY)

    h = _gat_layer(x, src, dst, W1, a_s1, a_d1, b1, g1, be1, 448, H, True)
    h = _gat_layer(h, src, dst, W2, a_s2, a_d2, b2, g2, be2, 384, H, True)
    h = _gat_layer(h, src, dst, W3, a_s3, a_d3, b3, g3, be3, 256, H, True)
    w4p = jnp.pad(W4, ((0, 0), (0, 1024 - W4.shape[1])))
    as4p = jnp.pad(a_s4, ((0, 0), (0, 1024 - a_s4.shape[1])))
    ad4p = jnp.pad(a_d4, ((0, 0), (0, 1024 - a_d4.shape[1])))
    b4p = jnp.pad(b4, (0, 1024 - b4.shape[0]))
    out = _gat_layer(h, src, dst, w4p, as4p, ad4p, b4p, b4p, b4p,
                     1024, 1, False)
    return out[:, :W4.shape[1]]


# speedup vs baseline: 5.1819x; 1.1574x over previous
import functools
import jax
import jax.numpy as jnp
from jax.experimental import pallas as pl
from jax.experimental.pallas import tpu as pltpu
from jax.experimental.pallas import tpu_sc as plsc

H = 8
N_NODES = 10000
NB = 1000
NB_POST = 400
E_PAD = 172032
EB = 168
EBS = 128
DUMMY = N_NODES
NEG_BIG = -1e30
STRIPE = 640
NROWS = 10240


def _proj_kernel(x_ref, w_ref, asf_ref, adf_ref, h_ref, asrc_ref, adst_ref,
                 *, d_head):
    c2 = pl.program_id(1)
    acc = jnp.dot(x_ref[...], w_ref[...], preferred_element_type=jnp.float32)
    s_up = jnp.zeros((x_ref.shape[0], 128), jnp.float32)
    d_up = jnp.zeros((x_ref.shape[0], 128), jnp.float32)
    for i in range(4):
        col = c2 * 4 + i
        hd = (col * 64) // d_head
        mask = (jax.lax.broadcasted_iota(jnp.int32, (1, 128), 1) == hd
                ).astype(jnp.float32)
        blk = acc[:, i * 64:(i + 1) * 64]
        sv = jnp.sum(blk * asf_ref[0, i * 64:(i + 1) * 64][None, :], axis=1)
        dv = jnp.sum(blk * adf_ref[0, i * 64:(i + 1) * 64][None, :], axis=1)
        s_up = s_up + sv[:, None] * mask
        d_up = d_up + dv[:, None] * mask
    for i in range(2):
        h_ref[i] = acc[:, i * 128:(i + 1) * 128]

    @pl.when(c2 == 0)
    def _():
        asrc_ref[...] = jnp.zeros_like(asrc_ref)
        adst_ref[...] = jnp.zeros_like(adst_ref)

    asrc_ref[...] += s_up
    adst_ref[...] += d_up


def _proj(x, w, a_s, a_d, d_head):
    din = x.shape[1]
    hd_total = w.shape[1]
    c2 = hd_total // 128
    asf = a_s.reshape(1, hd_total)
    adf = a_d.reshape(1, hd_total)
    grid = (N_NODES // NB, c2 // 2)
    return pl.pallas_call(
        functools.partial(_proj_kernel, d_head=d_head),
        grid=grid,
        in_specs=[
            pl.BlockSpec((NB, din), lambda n, c: (n, 0)),
            pl.BlockSpec((din, 256), lambda n, c: (0, c)),
            pl.BlockSpec((1, 256), lambda n, c: (0, c)),
            pl.BlockSpec((1, 256), lambda n, c: (0, c)),
        ],
        out_specs=[
            pl.BlockSpec((2, NB, 128), lambda n, c: (c, n, 0)),
            pl.BlockSpec((NB, 128), lambda n, c: (n, 0)),
            pl.BlockSpec((NB, 128), lambda n, c: (n, 0)),
        ],
        out_shape=[
            jax.ShapeDtypeStruct((c2, N_NODES, 128), jnp.float32),
            jax.ShapeDtypeStruct((N_NODES, 128), jnp.float32),
            jax.ShapeDtypeStruct((N_NODES, 128), jnp.float32),
        ],
    )(x, w, asf, adf)


def _post_kernel(outw_ref, den_ref, b_ref, g_ref, be_ref, y_ref,
                 *, d_head, n_heads, do_ln):
    den = den_ref[0, :, :n_heads] + den_ref[1, :, :n_heads] + 1e-16
    nb = outw_ref.shape[0]
    hd_total = y_ref.shape[1]
    den_e = jnp.broadcast_to(den[:, :, None], (nb, n_heads, d_head)
                             ).reshape(nb, hd_total)
    z = outw_ref[...] / den_e + b_ref[0][None, :]
    if do_ln:
        z = jnp.maximum(z, 0.0)
        mu = jnp.mean(z, axis=1, keepdims=True)
        var = jnp.mean((z - mu) ** 2, axis=1, keepdims=True)
        z = (z - mu) * jax.lax.rsqrt(var + 1e-5) * g_ref[0][None, :] \
            + be_ref[0][None, :]
    y_ref[...] = z


def _post(outw, denp, b, g, be, d_head, n_heads, do_ln):
    hd_total = outw.shape[1]
    return pl.pallas_call(
        functools.partial(_post_kernel, d_head=d_head, n_heads=n_heads,
                          do_ln=do_ln),
        grid=(N_NODES // NB_POST,),
        in_specs=[
            pl.BlockSpec((NB_POST, hd_total), lambda n: (n, 0)),
            pl.BlockSpec((2, NB_POST, 128), lambda n: (0, n, 0)),
            pl.BlockSpec((1, hd_total), lambda n: (0, 0)),
            pl.BlockSpec((1, hd_total), lambda n: (0, 0)),
            pl.BlockSpec((1, hd_total), lambda n: (0, 0)),
        ],
        out_specs=pl.BlockSpec((NB_POST, hd_total), lambda n: (n, 0)),
        out_shape=jax.ShapeDtypeStruct((N_NODES, hd_total), jnp.float32),
    )(outw, denp, b.reshape(1, -1), g.reshape(1, -1), be.reshape(1, -1))


def _sc_mesh():
    return plsc.VectorSubcoreMesh(core_axis_name="c", subcore_axis_name="s")


def _sc_softmax(asrc, adst_p, src, dst):
    per_w = E_PAD // 32
    n_batches = per_w // EBS

    @functools.partial(
        pl.kernel, mesh=_sc_mesh(),
        out_type=[
            jax.ShapeDtypeStruct((E_PAD * 16,), jnp.float32),
            jax.ShapeDtypeStruct((2 * NROWS, 128), jnp.float32),
        ],
        scratch_types=[
            pltpu.VMEM((EBS,), jnp.int32),
            pltpu.VMEM((1, EBS), jnp.int32),
            pltpu.VMEM((EBS, 128), jnp.float32),
            pltpu.VMEM((EBS, 128), jnp.float32),
            pltpu.VMEM((EBS * 16,), jnp.float32),
            pltpu.VMEM_SHARED((NROWS, 128), jnp.float32),
        ],
    )
    def k(asrc_hbm, adst_hbm, src_hbm, dst_hbm, zden_hbm, ew_hbm, denp_hbm,
          sidx, didx, arows, brows, ewf, den_sh):
        core = jax.lax.axis_index("c")
        sub = jax.lax.axis_index("s")
        wid = sub * 2 + core
        pltpu.sync_copy(zden_hbm, den_sh.at[pl.ds(sub * STRIPE, STRIPE)])
        plsc.subcore_barrier()
        base_w = wid * per_w

        @pl.loop(0, n_batches)
        def _(b):
            base = base_w + b * EBS
            pltpu.sync_copy(src_hbm.at[pl.ds(base, EBS)], sidx)
            pltpu.sync_copy(dst_hbm.at[pl.ds(base, EBS)], didx.at[0])
            pltpu.sync_copy(asrc_hbm.at[sidx], arows)
            pltpu.sync_copy(adst_hbm.at[didx.at[0]], brows)

            @pl.loop(0, EBS)
            def _(i):
                v = arows[i, pl.ds(0, 16)] + brows[i, pl.ds(0, 16)]
                v = jnp.where(v > 0, v, 0.2 * v)
                v = jnp.exp(v)
                arows[i, pl.ds(0, 16)] = v
                ewf[pl.ds(i * 16, 16)] = v

            pltpu.sync_copy(ewf, ew_hbm.at[pl.ds(base * 16, EBS * 16)])
            pltpu.sync_copy(arows, den_sh.at[didx.at[0]], add=True)

        plsc.subcore_barrier()
        pltpu.sync_copy(
            den_sh.at[pl.ds(sub * STRIPE, STRIPE)],
            denp_hbm.at[pl.ds(core * NROWS + sub * STRIPE, STRIPE)])

    zden = jnp.zeros((STRIPE, 128), jnp.float32)
    ew, denp = k(asrc, adst_p, src, dst, zden)
    return ew, denp.reshape(2, NROWS, 128)


def _sc_spmm(h2d, src, dst, ew, c2, d_head):
    per_s = E_PAD // 16
    nb2 = per_s // EB
    half = c2 // 2
    hd_total = c2 * 128

    @functools.partial(
        pl.kernel, mesh=_sc_mesh(),
        out_type=jax.ShapeDtypeStruct((NROWS, hd_total), jnp.float32),
        scratch_types=[
            pltpu.VMEM((EB,), jnp.int32),
            pltpu.VMEM((EB,), jnp.int32),
            pltpu.VMEM((1, EB), jnp.int32),
            pltpu.VMEM((EB, 128), jnp.float32),
            pltpu.VMEM((EB, 128), jnp.float32),
            pltpu.VMEM((EB * 16,), jnp.float32),
            pltpu.VMEM_SHARED((NROWS, 128), jnp.float32),
            pltpu.SemaphoreType.DMA,
            pltpu.SemaphoreType.DMA,
        ],
    )
    def k(h_hbm, src_hbm, dst_hbm, ew_hbm, zacc_hbm, out_hbm,
          gidx_a, gidx_b, didx, rows_a, rows_b, ewbf, acc, sem_a, sem_b):
        core = jax.lax.axis_index("c")
        sub = jax.lax.axis_index("s")

        def start_gather(b, cbase, gidx, rows, sem):
            base = sub * per_s + b * EB
            pltpu.sync_copy(src_hbm.at[pl.ds(base, EB)], gidx)

            @pl.loop(0, EB, step=16)
            def _(i):
                gidx[pl.ds(i, 16)] = gidx[pl.ds(i, 16)] + cbase

            pltpu.async_copy(h_hbm.at[gidx], rows, sem)

        def process(b, hd_lo, hd_hi, gidx, rows, sem):
            base = sub * per_s + b * EB
            pltpu.sync_copy(dst_hbm.at[pl.ds(base, EB)], didx.at[0])
            pltpu.sync_copy(ew_hbm.at[pl.ds(base * 16, EB * 16)], ewbf)
            pltpu.make_async_copy(h_hbm.at[gidx], rows, sem).wait()

            @pl.loop(0, EB)
            def _(i):
                wv = ewbf[pl.ds(i * 16, 16)]
                wlo = wv[jnp.full((16,), hd_lo, jnp.int32)]
                whi = wv[jnp.full((16,), hd_hi, jnp.int32)]
                for jc in range(4):
                    sl = pl.ds(jc * 16, 16)
                    rows[i, sl] = rows[i, sl] * wlo
                for jc in range(4, 8):
                    sl = pl.ds(jc * 16, 16)
                    rows[i, sl] = rows[i, sl] * whi

            pltpu.sync_copy(rows, acc.at[didx.at[0]], add=True)

        @pl.loop(0, half)
        def _(j):
            c = j * 2 + core
            hd_lo = (c * 128) // d_head
            hd_hi = (c * 128 + 64) // d_head
            cbase = c * N_NODES
            pltpu.sync_copy(zacc_hbm, acc.at[pl.ds(sub * STRIPE, STRIPE)])
            plsc.subcore_barrier()
            start_gather(0, cbase, gidx_a, rows_a, sem_a)

            @pl.loop(0, nb2, step=2)
            def _(b):
                start_gather(b + 1, cbase, gidx_b, rows_b, sem_b)
                process(b, hd_lo, hd_hi, gidx_a, rows_a, sem_a)

                @pl.when(b + 2 < nb2)
                def _():
                    start_gather(b + 2, cbase, gidx_a, rows_a, sem_a)

                process(b + 1, hd_lo, hd_hi, gidx_b, rows_b, sem_b)

            plsc.subcore_barrier()
            pltpu.sync_copy(
                acc.at[pl.ds(sub * STRIPE, STRIPE)],
                out_hbm.at[pl.ds(sub * STRIPE, STRIPE), pl.ds(c * 128, 128)])

    zacc = jnp.zeros((STRIPE, 128), jnp.float32)
    return k(h2d, src, dst, ew, zacc)


def _gat_layer(x, src, dst, w, a_s, a_d, b, g, be, d_head, n_heads, do_ln):
    h3d, asrc, adst = _proj(x, w, a_s, a_d, d_head)
    c2 = h3d.shape[0]
    adst_p = jnp.concatenate(
        [adst, jnp.full((16, 128), NEG_BIG, jnp.float32)])
    ew, denp = _sc_softmax(asrc, adst_p, src, dst)
    h2d = h3d.reshape(c2 * N_NODES, 128)
    outw = _sc_spmm(h2d, src, dst, ew, c2, d_head)
    return _post(outw, denp, b, g, be, d_head, n_heads, do_ln)


def kernel(x, edge_index, W1, a_s1, a_d1, b1, g1, be1, W2, a_s2, a_d2, b2,
           g2, be2, W3, a_s3, a_d3, b3, g3, be3, W4, a_s4, a_d4, b4):
    n = x.shape[0]
    loop = jnp.arange(n, dtype=edge_index.dtype)
    src = jnp.concatenate([edge_index[0], loop])
    dst = jnp.concatenate([edge_index[1], loop])
    e_real = src.shape[0]
    src = jnp.pad(src, (0, E_PAD - e_real))
    dst = jnp.pad(dst, (0, E_PAD - e_real), constant_values=DUMMY)

    h = _gat_layer(x, src, dst, W1, a_s1, a_d1, b1, g1, be1, 448, H, True)
    h = _gat_layer(h, src, dst, W2, a_s2, a_d2, b2, g2, be2, 384, H, True)
    h = _gat_layer(h, src, dst, W3, a_s3, a_d3, b3, g3, be3, 256, H, True)
    w4p = jnp.pad(W4, ((0, 0), (0, 1024 - W4.shape[1])))
    as4p = jnp.pad(a_s4, ((0, 0), (0, 1024 - a_s4.shape[1])))
    ad4p = jnp.pad(a_d4, ((0, 0), (0, 1024 - a_d4.shape[1])))
    b4p = jnp.pad(b4, (0, 1024 - b4.shape[0]))
    out = _gat_layer(h, src, dst, w4p, as4p, ad4p, b4p, b4p, b4p,
                     1024, 1, False)
    return out[:, :W4.shape[1]]
